# baseline (speedup 1.0000x reference)
import jax
import jax.numpy as jnp
from jax import lax
from jax.experimental import pallas as pl
from jax.experimental.pallas import tpu as pltpu
from jax.experimental.pallas import tpu_sc as plsc

D = 14336
NC, NS = 2, 16
NW = NC * NS
B = 512
BPW = B // NW
CH = 2
NB = 4
NCHUNK = BPW // CH


def _body(idx_hbm, table_hbm, out_hbm, idx_v, buf0, buf1, buf2, buf3,
          g0, g1, g2, g3, w0, w1, w2, w3):
    wid = lax.axis_index("s") * NC + lax.axis_index("c")
    base = wid * BPW
    pltpu.sync_copy(idx_hbm.at[wid], idx_v)
    bufs = (buf0, buf1, buf2, buf3)
    gsems = (g0, g1, g2, g3)
    wsems = (w0, w1, w2, w3)

    def gather(j, b):
        return pltpu.make_async_copy(
            table_hbm.at[idx_v.at[j]], bufs[b], gsems[b])

    def write(j, b):
        return pltpu.make_async_copy(
            bufs[b], out_hbm.at[pl.ds(base + j * CH, CH)], wsems[b])

    for j in range(NB):
        gather(j, j).start()
    nrounds = NCHUNK // NB
    for r in range(nrounds):
        j0 = r * NB
        for k in range(NB):
            gather(j0 + k, k).wait()
            write(j0 + k, k).start()
        if r + 1 < nrounds:
            for k in range(NB):
                write(j0 + k, k).wait()
                gather(j0 + k + NB, k).start()
    for k in range(NB):
        write(NCHUNK - NB + k, k).wait()


_gather_call = pl.kernel(
    _body,
    out_type=jax.ShapeDtypeStruct((B, D), jnp.float32),
    mesh=plsc.VectorSubcoreMesh(core_axis_name="c", subcore_axis_name="s"),
    scratch_types=(
        [pltpu.VMEM((NCHUNK, CH), jnp.int32)]
        + [pltpu.VMEM((CH, D), jnp.float32)] * NB
        + [pltpu.SemaphoreType.DMA] * (2 * NB)
    ),
)


def kernel(prefix, embedding_table):
    bsz, seq = prefix.shape
    idx = prefix.astype(jnp.int32).reshape(NW, NCHUNK, CH)
    out = _gather_call(idx, embedding_table)
    return out.reshape(bsz, seq, D)


# speedup vs baseline: 1.0550x; 1.0550x over previous
import jax
import jax.numpy as jnp
from jax import lax
from jax.experimental import pallas as pl
from jax.experimental.pallas import tpu as pltpu
from jax.experimental.pallas import tpu_sc as plsc

D = 14336
NC, NS = 2, 16
NW = NC * NS
B = 512
BPW = B // NW
CH = 2
NB = 4
NCHUNK = BPW // CH


def _body(idx_hbm, table_hbm, out_hbm, idx_v, buf0, buf1, buf2, buf3,
          g0, g1, g2, g3, w0, w1, w2, w3):
    wid = lax.axis_index("s") * NC + lax.axis_index("c")
    base = wid * BPW
    pltpu.sync_copy(idx_hbm.at[wid], idx_v)
    bufs = (buf0, buf1, buf2, buf3)
    gsems = (g0, g1, g2, g3)
    wsems = (w0, w1, w2, w3)

    def gather(j, b):
        return pltpu.make_async_copy(
            table_hbm.at[idx_v.at[j]], bufs[b], gsems[b])

    def write(j, b):
        return pltpu.make_async_copy(
            bufs[b], out_hbm.at[pl.ds(base + j * CH, CH)], wsems[b])

    for j in range(NB):
        gather(j, j).start()
    for j in range(NCHUNK):
        b = j % NB
        gather(j, b).wait()
        write(j, b).start()
        jj = j - 1
        if jj >= 0 and jj + NB < NCHUNK:
            write(jj, jj % NB).wait()
            gather(jj + NB, jj % NB).start()
    for j in range(NCHUNK - NB, NCHUNK):
        write(j, j % NB).wait()


_gather_call = pl.kernel(
    _body,
    out_type=jax.ShapeDtypeStruct((B, D), jnp.float32),
    mesh=plsc.VectorSubcoreMesh(core_axis_name="c", subcore_axis_name="s"),
    scratch_types=(
        [pltpu.VMEM((NCHUNK, CH), jnp.int32)]
        + [pltpu.VMEM((CH, D), jnp.float32)] * NB
        + [pltpu.SemaphoreType.DMA] * (2 * NB)
    ),
)


def kernel(prefix, embedding_table):
    bsz, seq = prefix.shape
    idx = prefix.astype(jnp.int32).reshape(NW, NCHUNK, CH)
    out = _gather_call(idx, embedding_table)
    return out.reshape(bsz, seq, D)
